# Initial kernel scaffold; baseline (speedup 1.0000x reference)
import jax
import jax.numpy as jnp
from jax.experimental import pallas as pl


def kernel(x_embed, x_other, tables, W1, b1, W2, b2, W3, b3):
    raise NotImplementedError("write your pallas kernel here")


# speedup vs baseline: 7.7933x; 7.7933x over previous
import functools

import jax
import jax.numpy as jnp
from jax import lax
from jax.experimental import pallas as pl
from jax.experimental.pallas import tpu as pltpu
from jax.experimental.pallas import tpu_sc as plsc

NW = 32
SUB = 128


@functools.lru_cache(maxsize=None)
def _make_gather(n_rows, n_fields, dim, chunk_rows):
    total = n_rows * n_fields
    per_w = total // NW
    ci = chunk_rows * n_fields
    n_chunks = per_w // ci
    n_sub = ci // SUB
    mesh = plsc.VectorSubcoreMesh(core_axis_name="c", subcore_axis_name="s")

    @functools.partial(
        pl.kernel,
        out_type=jax.ShapeDtypeStruct((total, dim), jnp.float32),
        mesh=mesh,
        compiler_params=pltpu.CompilerParams(use_tc_tiling_on_sc=False),
        scratch_types=[
            pltpu.VMEM((ci,), jnp.int32),
            pltpu.VMEM((ci, dim), jnp.float32),
            pltpu.SemaphoreType.DMA,
        ],
    )
    def gather(table_hbm, idx_hbm, out_hbm, idx_v, rows_v, sem):
        wid = lax.axis_index("s") * 2 + lax.axis_index("c")
        base = wid * per_w

        def chunk_body(c, _):
            off = base + c * ci
            pltpu.sync_copy(idx_hbm.at[pl.ds(off, ci)], idx_v)
            copies = []
            for j in range(n_sub):
                copies.append(pltpu.async_copy(
                    table_hbm.at[idx_v.at[pl.ds(j * SUB, SUB)]],
                    rows_v.at[pl.ds(j * SUB, SUB)],
                    sem,
                ))
            for cp in copies:
                cp.wait()
            pltpu.sync_copy(rows_v, out_hbm.at[pl.ds(off, ci)])
            return 0

        lax.fori_loop(0, n_chunks, chunk_body, 0)

    return gather


@functools.lru_cache(maxsize=None)
def _make_mlp(n_rows, d_emb, d_other, h1, h2, block_rows):
    def body(e_ref, xo_ref, w1a_ref, w1b_ref, b1_ref, w2_ref, b2_ref,
             w3_ref, b3_ref, o_ref):
        x = jnp.dot(e_ref[...], w1a_ref[...], preferred_element_type=jnp.float32)
        x += jnp.dot(xo_ref[...], w1b_ref[...], preferred_element_type=jnp.float32)
        x = jnp.maximum(x + b1_ref[...], 0.0)
        x = jnp.dot(x, w2_ref[...], preferred_element_type=jnp.float32)
        x = jnp.maximum(x + b2_ref[...], 0.0)
        o_ref[...] = (jnp.dot(x, w3_ref[...], preferred_element_type=jnp.float32)
                      + b3_ref[...])

    rep = lambda i: (0, 0)
    return pl.pallas_call(
        body,
        grid=(n_rows // block_rows,),
        in_specs=[
            pl.BlockSpec((block_rows, d_emb), lambda i: (i, 0)),
            pl.BlockSpec((block_rows, d_other), lambda i: (i, 0)),
            pl.BlockSpec((d_emb, h1), rep),
            pl.BlockSpec((d_other, h1), rep),
            pl.BlockSpec((1, h1), rep),
            pl.BlockSpec((h1, h2), rep),
            pl.BlockSpec((1, h2), rep),
            pl.BlockSpec((h2, 1), rep),
            pl.BlockSpec((1, 1), rep),
        ],
        out_specs=pl.BlockSpec((block_rows, 1), lambda i: (i, 0)),
        out_shape=jax.ShapeDtypeStruct((n_rows, 1), jnp.float32),
    )


def kernel(x_embed, x_other, tables, W1, b1, W2, b2, W3, b3):
    n_rows, n_fields = x_embed.shape
    n_tab, vocab, dim = tables.shape
    d_emb = n_fields * dim
    d_other = x_other.shape[1]
    h1, h2 = W2.shape

    idx_flat = (x_embed
                + jnp.arange(n_fields, dtype=jnp.int32) * vocab).reshape(-1)
    table_flat = tables.reshape(n_tab * vocab, dim)

    embs = _make_gather(n_rows, n_fields, dim, 128)(table_flat, idx_flat)
    embs = embs.reshape(n_rows, d_emb)

    mlp = _make_mlp(n_rows, d_emb, d_other, h1, h2, 1024)
    return mlp(embs, x_other,
               W1[:d_emb], W1[d_emb:], b1.reshape(1, h1),
               W2, b2.reshape(1, h2),
               W3, b3.reshape(1, 1))
